# baseline (speedup 1.0000x reference)
import jax
import jax.numpy as jnp
from jax import lax
from jax.experimental import pallas as pl
from jax.experimental.pallas import tpu as pltpu
from jax.experimental.pallas import tpu_sc as plsc

_N = 16384
_C = 1000
_K = int(_N * 0.2)

_BT = 2048
_NTCOL = 12288
_NBLK = _NTCOL // _BT
_NSCCOL = _N - _NTCOL

_NC, _NS, _L = 2, 16, 16
_NW = _NC * _NS
_CPT = _NSCCOL // _NW
_NG = _CPT // _L
_NFULL = 62
_TAIL = _C - _L
_IMIN = -2**31
_IMAXP = 0x7FFFFFFF

_GDN = lax.GatherDimensionNumbers(
    offset_dims=(), collapsed_slice_dims=(0,), start_index_map=(0,))


def _lgather(v, idx):
    return lax.gather(v, idx[:, None], _GDN, (1,),
                      mode=lax.GatherScatterMode.PROMISE_IN_BOUNDS)


def _sc_cols(xt_hbm, t_hbm, a_hbm, m_hbm, s_hbm, tv_hbm, av_hbm,
             xbuf, tbuf, abuf, obuf, sem):
    wid = lax.axis_index("s") * _NC + lax.axis_index("c")
    out0 = wid * _CPT
    col0 = _NTCOL + out0
    pltpu.sync_copy(t_hbm.at[pl.ds(col0, _CPT)], tbuf)
    pltpu.sync_copy(a_hbm, abuf)
    pltpu.async_copy(
        xt_hbm.at[pl.ds(0, _C), pl.ds(col0, _CPT)], xbuf, sem).wait()

    tgs = [tbuf[pl.ds(g * _L, _L)] for g in range(_NG)]
    ninf = jnp.full((_L,), -jnp.inf, jnp.float32)
    zv = jnp.zeros((_L,), jnp.float32)

    def max_body(c, carry):
        ms, tvs = carry
        ms, tvs = list(ms), list(tvs)
        csp = jnp.full((_L,), c, jnp.int32)
        for g in range(_NG):
            v = xbuf[c, pl.ds(g * _L, _L)]
            ms[g] = jnp.maximum(ms[g], v)
            tvs[g] = jnp.where(csp == tgs[g], v, tvs[g])
        return tuple(ms), tuple(tvs)

    mg, tvg = lax.fori_loop(0, _C, max_body,
                            ((ninf,) * _NG, (zv,) * _NG))

    def exp_body(c, accs):
        accs = list(accs)
        for g in range(_NG):
            e = jnp.exp(xbuf[c, pl.ds(g * _L, _L)] - mg[g])
            accs[g] = accs[g] + e
        return tuple(accs)

    sg = lax.fori_loop(0, _C, exp_body, ((zv,) * _NG))

    offsets = [c * _L for c in range(_NFULL)] + [_TAIL]
    for g in range(_NG):
        obuf[pl.ds(g * _L, _L)] = mg[g]
        obuf[pl.ds(_CPT + g * _L, _L)] = sg[g]
        obuf[pl.ds(2 * _CPT + g * _L, _L)] = tvg[g]
        avec = zv
        for aoff in offsets:
            av_v = abuf[pl.ds(aoff, _L)]
            idx = jnp.clip(tgs[g] - aoff, 0, _L - 1)
            hit = (tgs[g] >= aoff) & (tgs[g] < aoff + _L)
            avec = jnp.where(hit, _lgather(av_v, idx), avec)
        obuf[pl.ds(3 * _CPT + g * _L, _L)] = avec

    pltpu.sync_copy(obuf.at[pl.ds(0, _CPT)], m_hbm.at[pl.ds(out0, _CPT)])
    pltpu.sync_copy(obuf.at[pl.ds(_CPT, _CPT)], s_hbm.at[pl.ds(out0, _CPT)])
    pltpu.sync_copy(obuf.at[pl.ds(2 * _CPT, _CPT)],
                    tv_hbm.at[pl.ds(out0, _CPT)])
    pltpu.sync_copy(obuf.at[pl.ds(3 * _CPT, _CPT)],
                    av_hbm.at[pl.ds(out0, _CPT)])


_sc_call = pl.kernel(
    _sc_cols,
    out_type=[jax.ShapeDtypeStruct((_NSCCOL,), jnp.float32)] * 4,
    mesh=plsc.VectorSubcoreMesh(core_axis_name="c", subcore_axis_name="s"),
    compiler_params=pltpu.CompilerParams(use_tc_tiling_on_sc=True),
    scratch_types=[
        pltpu.VMEM((_C, _CPT), jnp.float32),
        pltpu.VMEM((_CPT,), jnp.int32),
        pltpu.VMEM((_C,), jnp.float32),
        pltpu.VMEM((4 * _CPT,), jnp.float32),
        pltpu.SemaphoreType.DMA,
    ],
)


def _tc_main(xt_ref, t_ref, a_ref, loss_ref):
    xb = xt_ref[...]
    tg = t_ref[...].reshape(1, _BT)
    al = a_ref[...]

    row = jax.lax.broadcasted_iota(jnp.int32, (_C, _BT), 0)
    oh = (row == tg).astype(jnp.float32)
    tval = jnp.sum(xb * oh, axis=0)
    aval = jnp.sum(al * oh, axis=0)

    m = jnp.max(xb, axis=0)
    s = jnp.sum(jnp.exp(xb - m[None, :]), axis=0)
    lp = tval - (m + jnp.log(s))
    p = jnp.exp(lp)
    omp = 1.0 - p
    loss_ref[...] = -aval * omp * omp * lp


def _f32_key(v):
    b = jax.lax.bitcast_convert_type(v, jnp.int32)
    return jnp.where(b >= 0, b, b ^ _IMAXP)


def _tc_fin(lb_ref, m_ref, s_ref, tv_ref, av_ref, out_ref):
    m = m_ref[...]
    s = s_ref[...]
    tv = tv_ref[...]
    av = av_ref[...]
    lp = tv - (m + jnp.log(s))
    p = jnp.exp(lp)
    omp = 1.0 - p
    loss_sc = -av * omp * omp * lp
    vals = jnp.concatenate([lb_ref[...], loss_sc])
    keys = _f32_key(vals)
    one = jnp.int32(1)

    def bit_step(b, tu):
        cand = tu | (one << (31 - b))
        cnt = jnp.sum((keys >= (cand ^ _IMIN)).astype(jnp.int32))
        return jnp.where(cnt >= _K, cand, tu)

    tu = jax.lax.fori_loop(0, 32, bit_step, jnp.int32(0))
    ti = tu ^ _IMIN
    tb = jnp.where(ti >= 0, ti, ti ^ _IMAXP)
    tau = jax.lax.bitcast_convert_type(tb, jnp.float32)
    gt = keys > ti
    cnt_gt = jnp.sum(gt.astype(jnp.int32))
    sum_gt = jnp.sum(jnp.where(gt, vals, 0.0))
    out_ref[0, 0] = (sum_gt + (_K - cnt_gt).astype(jnp.float32) * tau) / _K


def kernel(inputs, targets, alpha):
    xt = inputs.T
    a1 = alpha.reshape(-1)
    t3 = targets.reshape(_N // _BT, 1, _BT)
    m, s, tv, av = _sc_call(xt, targets, a1)
    loss_bot = pl.pallas_call(
        _tc_main,
        grid=(_NBLK,),
        in_specs=[
            pl.BlockSpec((_C, _BT), lambda i: (0, i)),
            pl.BlockSpec((1, 1, _BT), lambda i: (i, 0, 0)),
            pl.BlockSpec((_C, 1), lambda i: (0, 0)),
        ],
        out_specs=pl.BlockSpec((_BT,), lambda i: (i,)),
        out_shape=jax.ShapeDtypeStruct((_NTCOL,), jnp.float32),
    )(xt, t3, alpha)
    out = pl.pallas_call(
        _tc_fin,
        out_specs=pl.BlockSpec(memory_space=pltpu.SMEM),
        out_shape=jax.ShapeDtypeStruct((1, 1), jnp.float32),
    )(loss_bot, m, s, tv, av)
    return out[0, 0]


# speedup vs baseline: 1.0096x; 1.0096x over previous
import jax
import jax.numpy as jnp
from jax import lax
from jax.experimental import pallas as pl
from jax.experimental.pallas import tpu as pltpu
from jax.experimental.pallas import tpu_sc as plsc

_N = 16384
_C = 1000
_K = int(_N * 0.2)

_BT = 2048
_NTCOL = 12288
_NBLK = _NTCOL // _BT
_NSCCOL = _N - _NTCOL

_NC, _NS, _L = 2, 16, 16
_NW = _NC * _NS
_CPT = _NSCCOL // _NW
_NG = _CPT // _L
_NFULL = 62
_TAIL = _C - _L
_IMIN = -2**31
_IMAXP = 0x7FFFFFFF

_GDN = lax.GatherDimensionNumbers(
    offset_dims=(), collapsed_slice_dims=(0,), start_index_map=(0,))


def _lgather(v, idx):
    return lax.gather(v, idx[:, None], _GDN, (1,),
                      mode=lax.GatherScatterMode.PROMISE_IN_BOUNDS)


_CC = 256
_CCHUNKS = [(0, 256), (256, 256), (512, 256), (768, 232)]


def _sc_cols(xt_hbm, t_hbm, a_hbm, m_hbm, s_hbm, tv_hbm, av_hbm,
             xbuf0, xbuf1, tbuf, abuf, obuf, sem0, sem1):
    wid = lax.axis_index("s") * _NC + lax.axis_index("c")
    out0 = wid * _CPT
    col0 = _NTCOL + out0
    pltpu.sync_copy(t_hbm.at[pl.ds(col0, _CPT)], tbuf)
    pltpu.sync_copy(a_hbm, abuf)

    def start(k, buf, sem):
        c0, nc = _CCHUNKS[k]
        pltpu.async_copy(
            xt_hbm.at[pl.ds(c0, nc), pl.ds(col0, _CPT)],
            buf.at[pl.ds(0, nc)], sem)

    def wait(k, buf, sem):
        c0, nc = _CCHUNKS[k]
        pltpu.make_async_copy(
            xt_hbm.at[pl.ds(c0, nc), pl.ds(col0, _CPT)],
            buf.at[pl.ds(0, nc)], sem).wait()

    tgs = [tbuf[pl.ds(g * _L, _L)] for g in range(_NG)]
    ninf = jnp.full((_L,), -jnp.inf, jnp.float32)
    zv = jnp.zeros((_L,), jnp.float32)

    def proc_chunk(k, buf, carry):
        cbase, nc = _CCHUNKS[k]
        mrun, acc, tvs = carry

        def max_body(c, carry):
            ms, tvs = carry
            ms, tvs = list(ms), list(tvs)
            csp = jnp.full((_L,), c, jnp.int32) + cbase
            for g in range(_NG):
                v = buf[c, pl.ds(g * _L, _L)]
                ms[g] = jnp.maximum(ms[g], v)
                tvs[g] = jnp.where(csp == tgs[g], v, tvs[g])
            return tuple(ms), tuple(tvs)

        mloc, tvs = lax.fori_loop(0, nc, max_body, ((ninf,) * _NG, tvs))
        mnew = [jnp.maximum(mrun[g], mloc[g]) for g in range(_NG)]
        acc = [acc[g] * jnp.exp(mrun[g] - mnew[g]) for g in range(_NG)]

        def exp_body(c, accs):
            accs = list(accs)
            for g in range(_NG):
                e = jnp.exp(buf[c, pl.ds(g * _L, _L)] - mnew[g])
                accs[g] = accs[g] + e
            return tuple(accs)

        acc = lax.fori_loop(0, nc, exp_body, tuple(acc))
        return tuple(mnew), acc, tvs

    start(0, xbuf0, sem0)
    start(1, xbuf1, sem1)
    carry = ((ninf,) * _NG, (zv,) * _NG, (zv,) * _NG)
    wait(0, xbuf0, sem0)
    carry = proc_chunk(0, xbuf0, carry)
    start(2, xbuf0, sem0)
    wait(1, xbuf1, sem1)
    carry = proc_chunk(1, xbuf1, carry)
    start(3, xbuf1, sem1)
    wait(2, xbuf0, sem0)
    carry = proc_chunk(2, xbuf0, carry)
    wait(3, xbuf1, sem1)
    mg, sg, tvg = proc_chunk(3, xbuf1, carry)
    mg, sg, tvg = list(mg), list(sg), list(tvg)

    offsets = [c * _L for c in range(_NFULL)] + [_TAIL]
    for g in range(_NG):
        obuf[pl.ds(g * _L, _L)] = mg[g]
        obuf[pl.ds(_CPT + g * _L, _L)] = sg[g]
        obuf[pl.ds(2 * _CPT + g * _L, _L)] = tvg[g]
        avec = zv
        for aoff in offsets:
            av_v = abuf[pl.ds(aoff, _L)]
            idx = jnp.clip(tgs[g] - aoff, 0, _L - 1)
            hit = (tgs[g] >= aoff) & (tgs[g] < aoff + _L)
            avec = jnp.where(hit, _lgather(av_v, idx), avec)
        obuf[pl.ds(3 * _CPT + g * _L, _L)] = avec

    pltpu.sync_copy(obuf.at[pl.ds(0, _CPT)], m_hbm.at[pl.ds(out0, _CPT)])
    pltpu.sync_copy(obuf.at[pl.ds(_CPT, _CPT)], s_hbm.at[pl.ds(out0, _CPT)])
    pltpu.sync_copy(obuf.at[pl.ds(2 * _CPT, _CPT)],
                    tv_hbm.at[pl.ds(out0, _CPT)])
    pltpu.sync_copy(obuf.at[pl.ds(3 * _CPT, _CPT)],
                    av_hbm.at[pl.ds(out0, _CPT)])


_sc_call = pl.kernel(
    _sc_cols,
    out_type=[jax.ShapeDtypeStruct((_NSCCOL,), jnp.float32)] * 4,
    mesh=plsc.VectorSubcoreMesh(core_axis_name="c", subcore_axis_name="s"),
    compiler_params=pltpu.CompilerParams(use_tc_tiling_on_sc=True),
    scratch_types=[
        pltpu.VMEM((_CC, _CPT), jnp.float32),
        pltpu.VMEM((_CC, _CPT), jnp.float32),
        pltpu.VMEM((_CPT,), jnp.int32),
        pltpu.VMEM((_C,), jnp.float32),
        pltpu.VMEM((4 * _CPT,), jnp.float32),
        pltpu.SemaphoreType.DMA,
        pltpu.SemaphoreType.DMA,
    ],
)


def _tc_main(xt_ref, t_ref, a_ref, loss_ref):
    xb = xt_ref[...]
    tg = t_ref[...].reshape(1, _BT)
    al = a_ref[...]

    row = jax.lax.broadcasted_iota(jnp.int32, (_C, _BT), 0)
    oh = (row == tg).astype(jnp.float32)
    tval = jnp.sum(xb * oh, axis=0)
    aval = jnp.sum(al * oh, axis=0)

    m = jnp.max(xb, axis=0)
    s = jnp.sum(jnp.exp(xb - m[None, :]), axis=0)
    lp = tval - (m + jnp.log(s))
    p = jnp.exp(lp)
    omp = 1.0 - p
    loss_ref[...] = -aval * omp * omp * lp


def _f32_key(v):
    b = jax.lax.bitcast_convert_type(v, jnp.int32)
    return jnp.where(b >= 0, b, b ^ _IMAXP)


def _tc_fin(lb_ref, m_ref, s_ref, tv_ref, av_ref, out_ref):
    m = m_ref[...]
    s = s_ref[...]
    tv = tv_ref[...]
    av = av_ref[...]
    lp = tv - (m + jnp.log(s))
    p = jnp.exp(lp)
    omp = 1.0 - p
    loss_sc = -av * omp * omp * lp
    vals = jnp.concatenate([lb_ref[...], loss_sc])
    keys = _f32_key(vals)
    one = jnp.int32(1)

    def bit_step(b, tu):
        cand = tu | (one << (31 - b))
        cnt = jnp.sum((keys >= (cand ^ _IMIN)).astype(jnp.int32))
        return jnp.where(cnt >= _K, cand, tu)

    tu = jax.lax.fori_loop(0, 32, bit_step, jnp.int32(0))
    ti = tu ^ _IMIN
    tb = jnp.where(ti >= 0, ti, ti ^ _IMAXP)
    tau = jax.lax.bitcast_convert_type(tb, jnp.float32)
    gt = keys > ti
    cnt_gt = jnp.sum(gt.astype(jnp.int32))
    sum_gt = jnp.sum(jnp.where(gt, vals, 0.0))
    out_ref[0, 0] = (sum_gt + (_K - cnt_gt).astype(jnp.float32) * tau) / _K


def kernel(inputs, targets, alpha):
    xt = inputs.T
    a1 = alpha.reshape(-1)
    t3 = targets.reshape(_N // _BT, 1, _BT)
    m, s, tv, av = _sc_call(xt, targets, a1)
    loss_bot = pl.pallas_call(
        _tc_main,
        grid=(_NBLK,),
        in_specs=[
            pl.BlockSpec((_C, _BT), lambda i: (0, i)),
            pl.BlockSpec((1, 1, _BT), lambda i: (i, 0, 0)),
            pl.BlockSpec((_C, 1), lambda i: (0, 0)),
        ],
        out_specs=pl.BlockSpec((_BT,), lambda i: (i,)),
        out_shape=jax.ShapeDtypeStruct((_NTCOL,), jnp.float32),
    )(xt, t3, alpha)
    out = pl.pallas_call(
        _tc_fin,
        out_specs=pl.BlockSpec(memory_space=pltpu.SMEM),
        out_shape=jax.ShapeDtypeStruct((1, 1), jnp.float32),
    )(loss_bot, m, s, tv, av)
    return out[0, 0]
